# baseline (speedup 1.0000x reference)
import functools

import jax
import jax.numpy as jnp
from jax import lax
from jax.experimental import pallas as pl
from jax.experimental.pallas import tpu as pltpu
from jax.experimental.pallas import tpu_sc as plsc

VOCAB_SIZE = 100000
EMBED_DIM = 64
SEQ = 4096
LEN = 50
NUM_CORES = 2
NUM_SUBCORES = 16
NUM_WORKERS = NUM_CORES * NUM_SUBCORES
SPW = SEQ // NUM_WORKERS

_mesh = plsc.VectorSubcoreMesh(core_axis_name="c", subcore_axis_name="s")


@functools.partial(
    pl.kernel,
    out_type=jax.ShapeDtypeStruct((LEN, EMBED_DIM, SEQ), jnp.float32),
    mesh=_mesh,
    scratch_types=[
        pltpu.VMEM((LEN, SPW), jnp.int32),
        pltpu.VMEM((2, SPW, 128), jnp.float32),
        pltpu.VMEM((EMBED_DIM, SPW), jnp.float32),
        pltpu.SemaphoreType.DMA,
        pltpu.SemaphoreType.DMA,
    ],
    compiler_params=pltpu.CompilerParams(use_tc_tiling_on_sc=True, needs_layout_passes=False),
)
def _embed_sc(idx_hbm, table_hbm, out_hbm, idx_v, rows_v, tbuf, gsem, wsem):
    wid = lax.axis_index("s") * NUM_CORES + lax.axis_index("c")
    sb = wid * SPW
    pltpu.sync_copy(idx_hbm.at[:, pl.ds(sb, SPW)], idx_v)

    def start_gather(b, l):
        pltpu.async_copy(table_hbm.at[idx_v.at[l]], rows_v.at[b], gsem)

    def wait_gather(b, l):
        pltpu.make_async_copy(table_hbm.at[idx_v.at[l]], rows_v.at[b],
                              gsem).wait()

    lane = lax.iota(jnp.int32, 16)

    def transpose_and_write(b, l):
        def dbody(d, c):
            col = jnp.full((16,), d, jnp.int32)
            for k in range(SPW // 16):
                v = plsc.load_gather(rows_v.at[b], [k * 16 + lane, col])
                tbuf[d, pl.ds(k * 16, 16)] = v
            return c

        lax.fori_loop(0, EMBED_DIM, dbody, 0)
        pltpu.sync_copy(tbuf, out_hbm.at[l, :, pl.ds(sb, SPW)])

    start_gather(0, 0)

    def body(l, c):
        start_gather((l + 1) % 2, l + 1)
        wait_gather(l % 2, l)
        transpose_and_write(l % 2, l)
        return c

    lax.fori_loop(0, LEN - 1, body, 0)
    wait_gather((LEN - 1) % 2, LEN - 1)
    transpose_and_write((LEN - 1) % 2, LEN - 1)


def kernel(input_ids, table):
    ids_t = input_ids.astype(jnp.int32).T
    tablep = jnp.pad(table, ((0, 0), (0, 128 - EMBED_DIM)))
    out_t = _embed_sc(ids_t, tablep)
    embeds = out_t.transpose(2, 0, 1)
    return (embeds, embeds, embeds)
